# baseline (speedup 1.0000x reference)
import functools

import jax
import jax.numpy as jnp
from jax import lax
from jax.experimental import pallas as pl
from jax.experimental.pallas import tpu as pltpu
from jax.experimental.pallas import tpu_sc as plsc

_MAPPING = (0, 1, 1, 1, 1, 2, 2, 2, 2, 3, 3, 3, 3, 3, 4, 4, 4, 4, 5, 6, 7, 7, 7)
_DISTANCES = (0., 0., 1., 2., 3., 0., 1., 2., 3., 0., 1., 2., 3., 4., 0., 1., 2., 3., 0., 0., 0., 1., 2.)
_NCLS = 23
_NMAJ = 8
_E = 32

_NC = 2
_NS = 16
_NW = _NC * _NS
_TSTRIDE = 737
_CB = 8


@functools.partial(jax.jit, static_argnums=(3, 4))
def _sc_lookup(idx2d, vectors, bias, per_w, w):
    rows_total = idx2d.shape[0]
    nchunk = per_w // _CB
    mesh = plsc.VectorSubcoreMesh(core_axis_name="c", subcore_axis_name="s")

    @functools.partial(
        pl.kernel,
        mesh=mesh,
        compiler_params=pltpu.CompilerParams(needs_layout_passes=False),
        out_type=jax.ShapeDtypeStruct((rows_total, _E, w), jnp.float32),
        scratch_types=[
            pltpu.VMEM((_NMAJ, _E), jnp.float32),
            pltpu.VMEM((_NMAJ, _E), jnp.float32),
            pltpu.VMEM((16 * _TSTRIDE,), jnp.float32),
            pltpu.VMEM((_CB, w), jnp.int32),
            pltpu.VMEM((_CB, w), jnp.int32),
            pltpu.VMEM((_CB, w), jnp.int32),
            pltpu.VMEM((_CB, _E, w), jnp.float32),
            pltpu.VMEM((_CB, _E, w), jnp.float32),
            pltpu.VMEM((_CB, _E, w), jnp.float32),
            pltpu.SemaphoreType.DMA,
            pltpu.SemaphoreType.DMA,
            pltpu.SemaphoreType.DMA,
            pltpu.SemaphoreType.DMA,
            pltpu.SemaphoreType.DMA,
            pltpu.SemaphoreType.DMA,
        ],
    )
    def k(idx_hbm, vec_hbm, bias_hbm, out_hbm,
          vec_v, bias_v, table_v, idx0, idx1, idx2, ob0, ob1, ob2,
          sem0, sem1, sem2, isem0, isem1, isem2):
        wid = lax.axis_index("s") * _NC + lax.axis_index("c")
        lane = lax.iota(jnp.int32, 16)
        pltpu.sync_copy(vec_hbm, vec_v)
        pltpu.sync_copy(bias_hbm, bias_v)
        for c in range(_NCLS):
            m = _MAPPING[c]
            d = _DISTANCES[c]
            for h in range(_E // 16):
                val = bias_v[m, pl.ds(h * 16, 16)] + d * vec_v[m, pl.ds(h * 16, 16)]
                for rep in range(16):
                    plsc.store_scatter(
                        table_v, [lane + (rep * _TSTRIDE + c * _E + h * 16)], val)
        lane_rep = lane * _TSTRIDE
        base_row = wid * per_w
        obufs = (ob0, ob1, ob2)
        sems = (sem0, sem1, sem2)
        ibufs = (idx0, idx1, idx2)
        isems = (isem0, isem1, isem2)
        nbuf = 3
        ngrp = w // 16
        _D = 6

        def expand(idx_v, ob):
            def row_body(r, carry):
                bases = []
                for g in range(ngrp):
                    c16 = idx_v[r, pl.ds(g * 16, 16)]
                    bases.append(lane_rep + c16 * _E)
                pend = []
                for g in range(ngrp):
                    for e in range(_E):
                        v = plsc.load_gather(table_v, [bases[g] + e])
                        pend.append((g, e, v))
                        if len(pend) > _D:
                            g2, e2, v2 = pend.pop(0)
                            ob[r, e2, pl.ds(g2 * 16, 16)] = v2
                for g2, e2, v2 in pend:
                    ob[r, e2, pl.ds(g2 * 16, 16)] = v2
                return carry

            lax.fori_loop(0, _CB, row_body, 0, unroll=False)

        pltpu.async_copy(idx_hbm.at[pl.ds(base_row, _CB)], idx0, isem0)

        def outer(kk, carry):
            for b in range(nbuf):
                ob, sem = obufs[b], sems[b]
                ch = kk * nbuf + b
                row0 = base_row + ch * _CB

                @pl.when(ch + 1 < nchunk)
                def _prefetch():
                    pltpu.async_copy(
                        idx_hbm.at[pl.ds(row0 + _CB, _CB)],
                        ibufs[(b + 1) % nbuf], isems[(b + 1) % nbuf])

                pltpu.make_async_copy(
                    idx_hbm.at[pl.ds(base_row, _CB)], ibufs[b], isems[b]).wait()

                @pl.when(kk > 0)
                def _drain():
                    pltpu.make_async_copy(
                        out_hbm.at[pl.ds(base_row, _CB)], ob, sem).wait()

                expand(ibufs[b], ob)
                pltpu.async_copy(ob, out_hbm.at[pl.ds(row0, _CB)], sem)
            return carry

        lax.fori_loop(0, nchunk // nbuf, outer, 0, unroll=False)
        for b in range(nbuf):
            pltpu.make_async_copy(
                out_hbm.at[pl.ds(base_row, _CB)], obufs[b], sems[b]).wait()

    return k(idx2d, vectors, bias)


def kernel(input, vectors, bias):
    shape = input.shape
    w = shape[-1]
    rows_total = input.size // w
    assert w % 16 == 0 and rows_total % (_NW * _CB) == 0
    idx2d = input.reshape(rows_total, w).astype(jnp.int32)
    per_w = rows_total // _NW
    out = _sc_lookup(idx2d, vectors, bias, per_w, w)
    out = out.reshape(*shape[:-1], _E, w)
    perm = list(range(len(shape) - 1)) + [len(shape), len(shape) - 1]
    return out.transpose(*perm)


# speedup vs baseline: 1.0251x; 1.0251x over previous
import functools

import jax
import jax.numpy as jnp
from jax import lax
from jax.experimental import pallas as pl
from jax.experimental.pallas import tpu as pltpu
from jax.experimental.pallas import tpu_sc as plsc

_MAPPING = (0, 1, 1, 1, 1, 2, 2, 2, 2, 3, 3, 3, 3, 3, 4, 4, 4, 4, 5, 6, 7, 7, 7)
_DISTANCES = (0., 0., 1., 2., 3., 0., 1., 2., 3., 0., 1., 2., 3., 4., 0., 1., 2., 3., 0., 0., 0., 1., 2.)
_NCLS = 23
_NMAJ = 8
_E = 32

_NC = 2
_NS = 16
_NW = _NC * _NS
_TSTRIDE = 737
_CB = 8


@functools.partial(jax.jit, static_argnums=(3, 4))
def _sc_lookup(idx2d, vectors, bias, per_w, w):
    rows_total = idx2d.shape[0]
    nchunk = per_w // _CB
    mesh = plsc.VectorSubcoreMesh(core_axis_name="c", subcore_axis_name="s")

    @functools.partial(
        pl.kernel,
        mesh=mesh,
        compiler_params=pltpu.CompilerParams(needs_layout_passes=False),
        out_type=jax.ShapeDtypeStruct((rows_total, _E, w), jnp.float32),
        scratch_types=[
            pltpu.VMEM((_NMAJ, _E), jnp.float32),
            pltpu.VMEM((_NMAJ, _E), jnp.float32),
            pltpu.VMEM((16 * _TSTRIDE,), jnp.float32),
            pltpu.VMEM((_CB, w), jnp.int32),
            pltpu.VMEM((_CB, w), jnp.int32),
            pltpu.VMEM((_CB, _E, w), jnp.float32),
            pltpu.VMEM((_CB, _E, w), jnp.float32),
            pltpu.SemaphoreType.DMA,
            pltpu.SemaphoreType.DMA,
            pltpu.SemaphoreType.DMA,
            pltpu.SemaphoreType.DMA,
        ],
    )
    def k(idx_hbm, vec_hbm, bias_hbm, out_hbm,
          vec_v, bias_v, table_v, idx0, idx1, ob0, ob1,
          sem0, sem1, isem0, isem1):
        wid = lax.axis_index("s") * _NC + lax.axis_index("c")
        lane = lax.iota(jnp.int32, 16)
        base_row = wid * per_w
        pltpu.async_copy(idx_hbm.at[pl.ds(base_row, _CB)], idx0, isem0)
        cp_vec = pltpu.async_copy(vec_hbm, vec_v, sem0)
        cp_bias = pltpu.async_copy(bias_hbm, bias_v, sem1)
        cp_vec.wait()
        cp_bias.wait()
        for c in range(_NCLS):
            m = _MAPPING[c]
            d = _DISTANCES[c]
            for h in range(_E // 16):
                val = bias_v[m, pl.ds(h * 16, 16)] + d * vec_v[m, pl.ds(h * 16, 16)]
                for rep in range(16):
                    plsc.store_scatter(
                        table_v, [lane + (rep * _TSTRIDE + c * _E + h * 16)], val)
        lane_rep = lane * _TSTRIDE
        obufs = (ob0, ob1)
        sems = (sem0, sem1)
        ibufs = (idx0, idx1)
        isems = (isem0, isem1)
        ngrp = w // 16
        _D = 6

        def expand(idx_v, ob):
            def row_body(r, carry):
                bases = []
                for g in range(ngrp):
                    c16 = idx_v[r, pl.ds(g * 16, 16)]
                    bases.append(lane_rep + c16 * _E)
                pend = []
                for g in range(ngrp):
                    for e in range(_E):
                        v = plsc.load_gather(table_v, [bases[g] + e])
                        pend.append((g, e, v))
                        if len(pend) > _D:
                            g2, e2, v2 = pend.pop(0)
                            ob[r, e2, pl.ds(g2 * 16, 16)] = v2
                for g2, e2, v2 in pend:
                    ob[r, e2, pl.ds(g2 * 16, 16)] = v2
                return carry

            lax.fori_loop(0, _CB, row_body, 0, unroll=False)

        def outer(kk, carry):
            for b in range(2):
                ob, sem = obufs[b], sems[b]
                ch = kk * 2 + b
                row0 = base_row + ch * _CB

                @pl.when(ch + 1 < nchunk)
                def _prefetch():
                    pltpu.async_copy(
                        idx_hbm.at[pl.ds(row0 + _CB, _CB)], ibufs[1 - b],
                        isems[1 - b])

                pltpu.make_async_copy(
                    idx_hbm.at[pl.ds(base_row, _CB)], ibufs[b], isems[b]).wait()

                @pl.when(kk > 0)
                def _drain():
                    pltpu.make_async_copy(
                        out_hbm.at[pl.ds(base_row, _CB)], ob, sem).wait()

                expand(ibufs[b], ob)
                pltpu.async_copy(ob, out_hbm.at[pl.ds(row0, _CB)], sem)
            return carry

        lax.fori_loop(0, nchunk // 2, outer, 0, unroll=False)
        for b in range(2):
            pltpu.make_async_copy(
                out_hbm.at[pl.ds(base_row, _CB)], obufs[b], sems[b]).wait()

    return k(idx2d, vectors, bias)


def kernel(input, vectors, bias):
    shape = input.shape
    w = shape[-1]
    rows_total = input.size // w
    assert w % 16 == 0 and rows_total % (_NW * _CB) == 0
    idx2d = input.reshape(rows_total, w).astype(jnp.int32)
    per_w = rows_total // _NW
    out = _sc_lookup(idx2d, vectors, bias, per_w, w)
    out = out.reshape(*shape[:-1], _E, w)
    perm = list(range(len(shape) - 1)) + [len(shape), len(shape) - 1]
    return out.transpose(*perm)
